# Initial kernel scaffold; baseline (speedup 1.0000x reference)
import jax
import jax.numpy as jnp
from jax.experimental import pallas as pl


def kernel(tokens, connections, ram_memory):
    raise NotImplementedError("write your pallas kernel here")


# speedup vs baseline: 31.9916x; 31.9916x over previous
import functools

import jax
import jax.numpy as jnp
from jax import lax
from jax.experimental import pallas as pl
from jax.experimental.pallas import tpu as pltpu
from jax.experimental.pallas import tpu_sc as plsc

S = 128
BITS = 256
H = 8
NB = 12
TABLE = 1 << NB
THRESH = H // 2
N_TILES = 32
ROWS_PER_TILE = S // N_TILES


def _addr_body(tok_ref, tokT_ref, conn_ref, out_ref):
    tok = tok_ref[...].astype(jnp.float32)
    tokT = tokT_ref[...].astype(jnp.float32)
    iota_c = lax.broadcasted_iota(jnp.int32, (BITS, 1), 0)
    iota_r = lax.broadcasted_iota(jnp.int32, (1, BITS), 1)
    ii = lax.broadcasted_iota(jnp.int32, (S, S), 0)
    jj = lax.broadcasted_iota(jnp.int32, (S, S), 1)
    dd = ii - jj
    for h in range(H):
        wq = jnp.zeros((BITS, 1), jnp.float32)
        wk = jnp.zeros((1, BITS), jnp.float32)
        app = jnp.zeros((S, S), jnp.int32)
        for b in range(NB):
            c = conn_ref[h, b]
            wq = wq + jnp.where(iota_c == c, float(1 << b), 0.0)
            wk = wk + jnp.where(iota_r == c - BITS, float(1 << b), 0.0)
            app = app + jnp.where(
                jnp.logical_and(dd > c - 2 * BITS, c >= 2 * BITS), 1 << b, 0
            )
        aq = jnp.dot(tok, wq, preferred_element_type=jnp.float32)
        ak = jnp.dot(wk, tokT, preferred_element_type=jnp.float32)
        out_ref[h] = (
            aq.astype(jnp.int32) + ak.astype(jnp.int32) + app + h * TABLE
        )


_addr_call = pl.pallas_call(
    _addr_body,
    out_shape=jax.ShapeDtypeStruct((H, S, S), jnp.int32),
    in_specs=[
        pl.BlockSpec(memory_space=pltpu.VMEM),
        pl.BlockSpec(memory_space=pltpu.VMEM),
        pl.BlockSpec(memory_space=pltpu.SMEM),
    ],
    out_specs=pl.BlockSpec(memory_space=pltpu.VMEM),
)


def _votes_sc_body(addr_hbm, ram_hbm, votes_hbm, ram_v, addr_v, votes_v):
    cid = lax.axis_index("c")
    sid = lax.axis_index("s")
    wid = sid * 2 + cid
    base = wid * ROWS_PER_TILE
    pltpu.sync_copy(ram_hbm, ram_v)
    for h in range(H):
        pltpu.sync_copy(addr_hbm.at[h, pl.ds(base, ROWS_PER_TILE)], addr_v.at[h])
    for i in range(ROWS_PER_TILE):
        for jc in range(S // 16):
            acc = jnp.zeros((16,), jnp.int32)
            for h in range(H):
                idx = addr_v[h, i, pl.ds(jc * 16, 16)]
                acc = acc + plsc.load_gather(ram_v, [idx])
            votes_v[i, pl.ds(jc * 16, 16)] = acc
    pltpu.sync_copy(votes_v, votes_hbm.at[pl.ds(base, ROWS_PER_TILE)])


@functools.cache
def _votes_call():
    return pl.kernel(
        _votes_sc_body,
        out_type=jax.ShapeDtypeStruct((S, S), jnp.int32),
        mesh=plsc.VectorSubcoreMesh(core_axis_name="c", subcore_axis_name="s"),
        scratch_types=[
            pltpu.VMEM((H * TABLE,), jnp.int32),
            pltpu.VMEM((H, ROWS_PER_TILE, S), jnp.int32),
            pltpu.VMEM((ROWS_PER_TILE, S), jnp.int32),
        ],
        compiler_params=pltpu.CompilerParams(needs_layout_passes=False),
    )


def _agg_body(votes_ref, tok_ref, out_ref):
    votes = votes_ref[...]
    ii = lax.broadcasted_iota(jnp.int32, (S, S), 0)
    jj = lax.broadcasted_iota(jnp.int32, (S, S), 1)
    votes = jnp.where(jj <= ii, votes, 0)
    inc = votes >= THRESH
    count = jnp.sum(jnp.where(inc, 1, 0), axis=1, keepdims=True)
    rowmax = jnp.max(votes, axis=1, keepdims=True)
    firstmax = jnp.min(
        jnp.where(votes == rowmax, jj, S), axis=1, keepdims=True
    )
    inc_f = jnp.where(inc, 1.0, 0.0)
    fb_f = jnp.where(jj == firstmax, 1.0, 0.0)
    m = jnp.where(count == 0, fb_f, inc_f).astype(jnp.float32)
    tok = tok_ref[...].astype(jnp.float32)
    acc = jnp.dot(m, tok, preferred_element_type=jnp.float32)
    out_ref[...] = acc.astype(jnp.int32) & 1


_agg_call = pl.pallas_call(
    _agg_body,
    out_shape=jax.ShapeDtypeStruct((S, BITS), jnp.int32),
    in_specs=[
        pl.BlockSpec(memory_space=pltpu.VMEM),
        pl.BlockSpec(memory_space=pltpu.VMEM),
    ],
    out_specs=pl.BlockSpec(memory_space=pltpu.VMEM),
)


def kernel(tokens, connections, ram_memory):
    addr = _addr_call(tokens, tokens.T, connections)
    votes = _votes_call()(addr, ram_memory.reshape(-1))
    return _agg_call(votes, tokens)
